# baseline (speedup 1.0000x reference)
import functools

import jax
import jax.numpy as jnp
from jax import lax
from jax.experimental import pallas as pl
from jax.experimental.pallas import tpu as pltpu
from jax.experimental.pallas import tpu_sc as plsc

_B, _N, _F, _K = 4, 1024, 1024, 8
_EPS = 1e-5

_NC, _NS = 2, 16
_NW = _NC * _NS
_RW = (_B * _N) // _NW
_G = 4
_NCHUNK = _RW // _G
_RBLK = 256
_FP = _F // 2


def _topk_body(xyn_ref, xyt_ref, feat_ref, idx_ref, featT_ref):
    xyn = xyn_ref[0]
    xyt = xyt_ref[0]
    inner = 2.0 * lax.dot_general(
        xyn, xyt, (((1,), (0,)), ((), ())),
        preferred_element_type=jnp.float32)
    xx_row = jnp.sum(xyt * xyt, axis=0, keepdims=True)
    xx_col = jnp.sum(xyn * xyn, axis=1, keepdims=True)
    pw = (xx_row - inner) + xx_col

    colids = lax.broadcasted_iota(jnp.int32, (_RBLK, _N), 1)
    cur = pw
    picks = []
    for _ in range(_K):
        m = jnp.min(cur, axis=1, keepdims=True)
        cand = jnp.where(cur == m, colids, _N)
        sel = jnp.min(cand, axis=1, keepdims=True)
        picks.append(sel)
        cur = jnp.where(colids == sel, jnp.inf, cur)
    idx_ref[0] = jnp.concatenate(picks, axis=1)

    featT_ref[0] = jnp.transpose(feat_ref[0], (1, 0))


def _topk_call(xyn, xyt, feat):
    return pl.pallas_call(
        _topk_body,
        grid=(_B, _N // _RBLK),
        in_specs=[
            pl.BlockSpec((1, _RBLK, 2), lambda i, j: (i, j, 0)),
            pl.BlockSpec((1, 2, _N), lambda i, j: (i, 0, 0)),
            pl.BlockSpec((1, _RBLK, _F), lambda i, j: (i, j, 0)),
        ],
        out_specs=[
            pl.BlockSpec((1, _RBLK, _K), lambda i, j: (i, j, 0)),
            pl.BlockSpec((1, _F, _RBLK), lambda i, j: (i, 0, j)),
        ],
        out_shape=[
            jax.ShapeDtypeStruct((_B, _N, _K), jnp.int32),
            jax.ShapeDtypeStruct((_B, _N, _F), jnp.float32),
        ],
    )(xyn, xyt, feat)


def _gather_mean_body(table_hbm, idx_hbm, out_hbm,
                      idx_v, rows0, rows1, acc0, acc1,
                      sem0, sem1, ws0, ws1):
    wid = lax.axis_index("s") * _NC + lax.axis_index("c")
    pltpu.sync_copy(idx_hbm.at[wid], idx_v)
    base = wid * _RW
    rows = (rows0, rows1)
    acc = (acc0, acc1)
    gsem = (sem0, sem1)
    wsem = (ws0, ws1)

    lanes = lax.broadcasted_iota(jnp.int32, (16,), 0)
    hi_mask = jnp.full((16,), -65536, dtype=jnp.int32)

    pltpu.async_copy(table_hbm.at[idx_v.at[0]], rows0, sem0)
    pltpu.async_copy(table_hbm.at[idx_v.at[1]], rows1, sem1)

    def outer(cc, carry):
        for par in range(2):
            c = cc * 2 + par
            pltpu.make_async_copy(
                table_hbm.at[pl.ds(0, _G * _K)], rows[par], gsem[par]).wait()

            @pl.when(c >= 2)
            def _():
                pltpu.make_async_copy(
                    acc[par], out_hbm.at[pl.ds(0, _G)], wsem[par]).wait()

            @plsc.parallel_loop(0, _FP // 16, unroll=2)
            def _(jj):
                col = jj * 16
                col_lo = (col * 2) + (lanes * 2)
                col_hi = col_lo + 1
                for g in range(_G):
                    vb = [rows[par][g * _K + k, pl.ds(col, 16)]
                          for k in range(_K)]
                    lo = [plsc.bitcast(lax.shift_left(v, 16), jnp.float32)
                          for v in vb]
                    hi = [plsc.bitcast(jnp.bitwise_and(v, hi_mask),
                                       jnp.float32) for v in vb]
                    slo = ((lo[0] + lo[1]) + (lo[2] + lo[3])) + \
                          ((lo[4] + lo[5]) + (lo[6] + lo[7]))
                    shi = ((hi[0] + hi[1]) + (hi[2] + hi[3])) + \
                          ((hi[4] + hi[5]) + (hi[6] + hi[7]))
                    gsplat = jnp.full((16,), g, dtype=jnp.int32)
                    plsc.store_scatter(acc[par], [gsplat, col_lo],
                                       slo * (1.0 / _K))
                    plsc.store_scatter(acc[par], [gsplat, col_hi],
                                       shi * (1.0 / _K))

            @pl.when(c + 2 < _NCHUNK)
            def _():
                pltpu.async_copy(
                    table_hbm.at[idx_v.at[c + 2]], rows[par], gsem[par])

            pltpu.async_copy(
                acc[par], out_hbm.at[pl.ds(base + c * _G, _G)], wsem[par])
        return carry

    lax.fori_loop(0, _NCHUNK // 2, outer, 0)

    pltpu.make_async_copy(acc0, out_hbm.at[pl.ds(0, _G)], ws0).wait()
    pltpu.make_async_copy(acc1, out_hbm.at[pl.ds(0, _G)], ws1).wait()


@functools.lru_cache(maxsize=1)
def _make_gather_mean():
    return functools.partial(
        pl.kernel,
        mesh=plsc.VectorSubcoreMesh(core_axis_name="c", subcore_axis_name="s"),
        compiler_params=pltpu.CompilerParams(needs_layout_passes=False),
        out_type=jax.ShapeDtypeStruct((_B * _N, _F), jnp.float32),
        scratch_types=[
            pltpu.VMEM((_NCHUNK, _G * _K), jnp.int32),
            pltpu.VMEM((_G * _K, _FP), jnp.int32),
            pltpu.VMEM((_G * _K, _FP), jnp.int32),
            pltpu.VMEM((_G, _F), jnp.float32),
            pltpu.VMEM((_G, _F), jnp.float32),
            pltpu.SemaphoreType.DMA,
            pltpu.SemaphoreType.DMA,
            pltpu.SemaphoreType.DMA,
            pltpu.SemaphoreType.DMA,
        ],
    )(_gather_mean_body)


def _fused_mm_bn_body(feat_ref, m_ref, w_ref, bias_ref, g_ref, be_ref,
                      out_ref, trans_s, s1_s, s2_s):
    p = pl.program_id(0)
    bi = pl.program_id(1)
    j = pl.program_id(2)
    sl = pl.ds(j * _RBLK, _RBLK)

    @pl.when(p == 0)
    def _():
        lap = feat_ref[0] - m_ref[0]
        t = lax.dot_general(
            lap, w_ref[...], (((1,), (1,)), ((), ())),
            preferred_element_type=jnp.float32) + bias_ref[...]
        trans_s[bi, sl, :] = t
        rs1 = jnp.sum(t, axis=1, keepdims=True)
        rs2 = jnp.sum(t * t, axis=1, keepdims=True)

        @pl.when(bi == 0)
        def _():
            s1_s[sl] = rs1
            s2_s[sl] = rs2

        @pl.when(bi != 0)
        def _():
            s1_s[sl] = s1_s[sl] + rs1
            s2_s[sl] = s2_s[sl] + rs2

    @pl.when(p == 1)
    def _():
        cnt = float(_B * _F)
        mean = s1_s[sl] / cnt
        var = s2_s[sl] / cnt - mean * mean
        inv = lax.rsqrt(var + _EPS)
        t = trans_s[bi, sl, :]
        y = (t - mean) * inv * g_ref[...] + be_ref[...]
        out_ref[0] = feat_ref[0] + jnp.maximum(y, 0.0)


def _fused_mm_bn_call(feat, m, w, bias, gamma, beta):
    def m_map(p, i, j):
        return (jnp.where(p == 0, i, 0), jnp.where(p == 0, j, 0), 0)

    def out_map(p, i, j):
        return (jnp.where(p == 0, 0, i), jnp.where(p == 0, 0, j), 0)

    return pl.pallas_call(
        _fused_mm_bn_body,
        grid=(2, _B, _N // _RBLK),
        in_specs=[
            pl.BlockSpec((1, _RBLK, _F), lambda p, i, j: (i, j, 0)),
            pl.BlockSpec((1, _RBLK, _F), m_map),
            pl.BlockSpec((_F, _F), lambda p, i, j: (0, 0)),
            pl.BlockSpec((1, _F), lambda p, i, j: (0, 0)),
            pl.BlockSpec((_RBLK, 1), lambda p, i, j: (j, 0)),
            pl.BlockSpec((_RBLK, 1), lambda p, i, j: (j, 0)),
        ],
        out_specs=pl.BlockSpec((1, _RBLK, _F), out_map),
        out_shape=jax.ShapeDtypeStruct((_B, _N, _F), jnp.float32),
        scratch_shapes=[
            pltpu.VMEM((_B, _N, _F), jnp.float32),
            pltpu.VMEM((_N, 1), jnp.float32),
            pltpu.VMEM((_N, 1), jnp.float32),
        ],
    )(feat, m, w, bias, gamma, beta)


def kernel(xyz, feat, W, b, gamma, beta):
    xyn = xyz[:, :, :2]
    xyt = jnp.transpose(xyn, (0, 2, 1))
    idx, featT = _topk_call(xyn, xyt, feat)

    tbl = lax.bitcast_convert_type(
        featT.astype(jnp.bfloat16).reshape(_B * _N, _FP, 2), jnp.int32)

    cols = idx.reshape(_B, _K, _N // _K, _K).transpose(0, 2, 3, 1)
    cols = cols.reshape(_B, _N, _K)
    cols = cols + (jnp.arange(_B, dtype=jnp.int32) * _N)[:, None, None]
    idx_sc = cols.reshape(_NW, _NCHUNK, _G * _K)

    m = _make_gather_mean()(tbl, idx_sc)
    m = m.reshape(_B, _N, _F)

    return _fused_mm_bn_call(feat, m, W, b.reshape(1, _F),
                             gamma.reshape(_N, 1), beta.reshape(_N, 1))


# speedup vs baseline: 1.4108x; 1.4108x over previous
import functools

import jax
import jax.numpy as jnp
from jax import lax
from jax.experimental import pallas as pl
from jax.experimental.pallas import tpu as pltpu
from jax.experimental.pallas import tpu_sc as plsc

_B, _N, _F, _K = 4, 1024, 1024, 8
_EPS = 1e-5

_NC, _NS = 2, 16
_NW = _NC * _NS
_RW = (_B * _N) // _NW
_G = 4
_NCHUNK = _RW // _G
_RBLK = 256
_FP = _F // 2


def _topk_body(xyn_ref, xyt_ref, feat_ref, idx_ref, featT_ref):
    xyn = xyn_ref[0]
    xyt = xyt_ref[0]
    inner = 2.0 * lax.dot_general(
        xyn, xyt, (((1,), (0,)), ((), ())),
        preferred_element_type=jnp.float32)
    xx_row = jnp.sum(xyt * xyt, axis=0, keepdims=True)
    xx_col = jnp.sum(xyn * xyn, axis=1, keepdims=True)
    pw = (xx_row - inner) + xx_col

    colids = lax.broadcasted_iota(jnp.int32, (_RBLK, _N), 1)
    cur = pw
    picks = []
    for _ in range(_K):
        m = jnp.min(cur, axis=1, keepdims=True)
        cand = jnp.where(cur == m, colids, _N)
        sel = jnp.min(cand, axis=1, keepdims=True)
        picks.append(sel)
        cur = jnp.where(colids == sel, jnp.inf, cur)
    idx_ref[0] = jnp.concatenate(picks, axis=1)

    featT_ref[0] = jnp.transpose(feat_ref[0], (1, 0))


def _topk_call(xyn, xyt, feat):
    return pl.pallas_call(
        _topk_body,
        grid=(_B, _N // _RBLK),
        in_specs=[
            pl.BlockSpec((1, _RBLK, 2), lambda i, j: (i, j, 0)),
            pl.BlockSpec((1, 2, _N), lambda i, j: (i, 0, 0)),
            pl.BlockSpec((1, _RBLK, _F), lambda i, j: (i, j, 0)),
        ],
        out_specs=[
            pl.BlockSpec((1, _RBLK, _K), lambda i, j: (i, j, 0)),
            pl.BlockSpec((1, _F, _RBLK), lambda i, j: (i, 0, j)),
        ],
        out_shape=[
            jax.ShapeDtypeStruct((_B, _N, _K), jnp.int32),
            jax.ShapeDtypeStruct((_B, _N, _F), jnp.float32),
        ],
    )(xyn, xyt, feat)


def _gather_mean_body(table_hbm, idx_hbm, out_hbm,
                      idx_v, rows0, rows1, acc0, acc1,
                      sem0, sem1, ws0, ws1):
    wid = lax.axis_index("s") * _NC + lax.axis_index("c")
    pltpu.sync_copy(idx_hbm.at[wid], idx_v)
    base = wid * _RW
    rows = (rows0, rows1)
    acc = (acc0, acc1)
    gsem = (sem0, sem1)
    wsem = (ws0, ws1)

    hi_mask = jnp.full((16,), -65536, dtype=jnp.int32)

    pltpu.async_copy(table_hbm.at[idx_v.at[0]], rows0, sem0)
    pltpu.async_copy(table_hbm.at[idx_v.at[1]], rows1, sem1)

    def outer(cc, carry):
        for par in range(2):
            c = cc * 2 + par
            pltpu.make_async_copy(
                table_hbm.at[pl.ds(0, _G * _K)], rows[par], gsem[par]).wait()

            @pl.when(c >= 2)
            def _():
                pltpu.make_async_copy(
                    acc[par], out_hbm.at[pl.ds(0, _G)], wsem[par]).wait()

            @plsc.parallel_loop(0, _FP // 16, unroll=2)
            def _(jj):
                col = jj * 16
                for g in range(_G):
                    vb = [rows[par][g * _K + k, pl.ds(col, 16)]
                          for k in range(_K)]
                    lo = [plsc.bitcast(lax.shift_left(v, 16), jnp.float32)
                          for v in vb]
                    hi = [plsc.bitcast(jnp.bitwise_and(v, hi_mask),
                                       jnp.float32) for v in vb]
                    slo = ((lo[0] + lo[1]) + (lo[2] + lo[3])) + \
                          ((lo[4] + lo[5]) + (lo[6] + lo[7]))
                    shi = ((hi[0] + hi[1]) + (hi[2] + hi[3])) + \
                          ((hi[4] + hi[5]) + (hi[6] + hi[7]))
                    acc[par][g, pl.ds(col, 16)] = slo * (1.0 / _K)
                    acc[par][g, pl.ds(col + _FP, 16)] = shi * (1.0 / _K)

            @pl.when(c + 2 < _NCHUNK)
            def _():
                pltpu.async_copy(
                    table_hbm.at[idx_v.at[c + 2]], rows[par], gsem[par])

            pltpu.async_copy(
                acc[par], out_hbm.at[pl.ds(base + c * _G, _G)], wsem[par])
        return carry

    lax.fori_loop(0, _NCHUNK // 2, outer, 0)

    pltpu.make_async_copy(acc0, out_hbm.at[pl.ds(0, _G)], ws0).wait()
    pltpu.make_async_copy(acc1, out_hbm.at[pl.ds(0, _G)], ws1).wait()


@functools.lru_cache(maxsize=1)
def _make_gather_mean():
    return functools.partial(
        pl.kernel,
        mesh=plsc.VectorSubcoreMesh(core_axis_name="c", subcore_axis_name="s"),
        compiler_params=pltpu.CompilerParams(needs_layout_passes=False),
        out_type=jax.ShapeDtypeStruct((_B * _N, _F), jnp.float32),
        scratch_types=[
            pltpu.VMEM((_NCHUNK, _G * _K), jnp.int32),
            pltpu.VMEM((_G * _K, _FP), jnp.int32),
            pltpu.VMEM((_G * _K, _FP), jnp.int32),
            pltpu.VMEM((_G, _F), jnp.float32),
            pltpu.VMEM((_G, _F), jnp.float32),
            pltpu.SemaphoreType.DMA,
            pltpu.SemaphoreType.DMA,
            pltpu.SemaphoreType.DMA,
            pltpu.SemaphoreType.DMA,
        ],
    )(_gather_mean_body)


def _fused_mm_bn_body(feat_ref, m_ref, w_ref, bias_ref, g_ref, be_ref,
                      out_ref, trans_s, s1_s, s2_s):
    p = pl.program_id(0)
    bi = pl.program_id(1)
    j = pl.program_id(2)
    sl = pl.ds(j * _RBLK, _RBLK)

    @pl.when(p == 0)
    def _():
        lap = feat_ref[0] - m_ref[0]
        t = lax.dot_general(
            lap, w_ref[...], (((1,), (1,)), ((), ())),
            preferred_element_type=jnp.float32) + bias_ref[...]
        trans_s[bi, sl, :] = t
        rs1 = jnp.sum(t, axis=1, keepdims=True)
        rs2 = jnp.sum(t * t, axis=1, keepdims=True)

        @pl.when(bi == 0)
        def _():
            s1_s[sl] = rs1
            s2_s[sl] = rs2

        @pl.when(bi != 0)
        def _():
            s1_s[sl] = s1_s[sl] + rs1
            s2_s[sl] = s2_s[sl] + rs2

    @pl.when(p == 1)
    def _():
        cnt = float(_B * _F)
        mean = s1_s[sl] / cnt
        var = s2_s[sl] / cnt - mean * mean
        inv = lax.rsqrt(var + _EPS)
        t = trans_s[bi, sl, :]
        y = (t - mean) * inv * g_ref[...] + be_ref[...]
        out_ref[0] = feat_ref[0] + jnp.maximum(y, 0.0)


def _fused_mm_bn_call(feat, m, w, bias, gamma, beta):
    def m_map(p, i, j):
        return (jnp.where(p == 0, i, 0), jnp.where(p == 0, j, 0), 0)

    def out_map(p, i, j):
        return (jnp.where(p == 0, 0, i), jnp.where(p == 0, 0, j), 0)

    return pl.pallas_call(
        _fused_mm_bn_body,
        grid=(2, _B, _N // _RBLK),
        in_specs=[
            pl.BlockSpec((1, _RBLK, _F), lambda p, i, j: (i, j, 0)),
            pl.BlockSpec((1, _RBLK, _F), m_map),
            pl.BlockSpec((_F, _F), lambda p, i, j: (0, 0)),
            pl.BlockSpec((1, _F), lambda p, i, j: (0, 0)),
            pl.BlockSpec((_RBLK, 1), lambda p, i, j: (j, 0)),
            pl.BlockSpec((_RBLK, 1), lambda p, i, j: (j, 0)),
        ],
        out_specs=pl.BlockSpec((1, _RBLK, _F), out_map),
        out_shape=jax.ShapeDtypeStruct((_B, _N, _F), jnp.float32),
        scratch_shapes=[
            pltpu.VMEM((_B, _N, _F), jnp.float32),
            pltpu.VMEM((_N, 1), jnp.float32),
            pltpu.VMEM((_N, 1), jnp.float32),
        ],
    )(feat, m, w, bias, gamma, beta)


def kernel(xyz, feat, W, b, gamma, beta):
    xyn = xyz[:, :, :2]
    xyt = jnp.transpose(xyn, (0, 2, 1))
    idx, featT = _topk_call(xyn, xyt, feat)

    f16 = featT.astype(jnp.bfloat16).reshape(_B * _N, _F)
    tbl = lax.bitcast_convert_type(
        jnp.stack([f16[:, :_FP], f16[:, _FP:]], axis=-1), jnp.int32)

    cols = idx.reshape(_B, _K, _N // _K, _K).transpose(0, 2, 3, 1)
    cols = cols.reshape(_B, _N, _K)
    cols = cols + (jnp.arange(_B, dtype=jnp.int32) * _N)[:, None, None]
    idx_sc = cols.reshape(_NW, _NCHUNK, _G * _K)

    m = _make_gather_mean()(tbl, idx_sc)
    m = m.reshape(_B, _N, _F)

    return _fused_mm_bn_call(feat, m, W, b.reshape(1, _F),
                             gamma.reshape(_N, 1), beta.reshape(_N, 1))


# speedup vs baseline: 1.6894x; 1.1975x over previous
import functools

import jax
import jax.numpy as jnp
from jax import lax
from jax.experimental import pallas as pl
from jax.experimental.pallas import tpu as pltpu
from jax.experimental.pallas import tpu_sc as plsc

_B, _N, _F, _K = 4, 1024, 1024, 8
_EPS = 1e-5

_NC, _NS = 2, 16
_NW = _NC * _NS
_RW = (_B * _N) // _NW
_G = 8
_NCHUNK = _RW // _G
_RBLK = 256
_FP = _F // 2


def _topk_body(xyn_ref, xyt_ref, feat_ref, idx_ref, featT_ref):
    xyn = xyn_ref[0]
    xyt = xyt_ref[0]
    inner = 2.0 * lax.dot_general(
        xyn, xyt, (((1,), (0,)), ((), ())),
        preferred_element_type=jnp.float32)
    xx_row = jnp.sum(xyt * xyt, axis=0, keepdims=True)
    xx_col = jnp.sum(xyn * xyn, axis=1, keepdims=True)
    pw = (xx_row - inner) + xx_col

    colids = lax.broadcasted_iota(jnp.int32, (_RBLK, _N), 1)
    cur = pw
    picks = []
    for _ in range(_K):
        sel = jnp.argmin(cur, axis=1).astype(jnp.int32)[:, None]
        picks.append(sel)
        cur = jnp.where(colids == sel, jnp.inf, cur)
    idx_ref[0] = jnp.concatenate(picks, axis=1)

    featT_ref[0] = jnp.transpose(feat_ref[0], (1, 0)).astype(jnp.bfloat16)


def _topk_call(xyn, xyt, feat):
    return pl.pallas_call(
        _topk_body,
        grid=(_B, _N // _RBLK),
        in_specs=[
            pl.BlockSpec((1, _RBLK, 2), lambda i, j: (i, j, 0)),
            pl.BlockSpec((1, 2, _N), lambda i, j: (i, 0, 0)),
            pl.BlockSpec((1, _RBLK, _F), lambda i, j: (i, j, 0)),
        ],
        out_specs=[
            pl.BlockSpec((1, _RBLK, _K), lambda i, j: (i, j, 0)),
            pl.BlockSpec((1, _F, _RBLK), lambda i, j: (i, 0, j)),
        ],
        out_shape=[
            jax.ShapeDtypeStruct((_B, _N, _K), jnp.int32),
            jax.ShapeDtypeStruct((_B, _N, _F), jnp.bfloat16),
        ],
    )(xyn, xyt, feat)


def _gather_mean_body(table_hbm, idx_hbm, out_hbm,
                      idx_v, rows0, rows1, acc0, acc1,
                      sem0, sem1, ws0, ws1):
    wid = lax.axis_index("s") * _NC + lax.axis_index("c")
    pltpu.sync_copy(idx_hbm.at[wid], idx_v)
    base = wid * _RW
    rows = (rows0, rows1)
    acc = (acc0, acc1)
    gsem = (sem0, sem1)
    wsem = (ws0, ws1)

    hi_mask = jnp.full((16,), -65536, dtype=jnp.int32)

    pltpu.async_copy(table_hbm.at[idx_v.at[0]], rows0, sem0)
    pltpu.async_copy(table_hbm.at[idx_v.at[1]], rows1, sem1)

    def outer(cc, carry):
        for par in range(2):
            c = cc * 2 + par
            pltpu.make_async_copy(
                table_hbm.at[pl.ds(0, _G * _K)], rows[par], gsem[par]).wait()

            @pl.when(c >= 2)
            def _():
                pltpu.make_async_copy(
                    acc[par], out_hbm.at[pl.ds(0, _G)], wsem[par]).wait()

            @plsc.parallel_loop(0, _FP // 16, unroll=2)
            def _(jj):
                col = jj * 16
                for g in range(_G):
                    vb = [rows[par][g * _K + k, pl.ds(col, 16)]
                          for k in range(_K)]
                    lo = [plsc.bitcast(lax.shift_left(v, 16), jnp.float32)
                          for v in vb]
                    hi = [plsc.bitcast(jnp.bitwise_and(v, hi_mask),
                                       jnp.float32) for v in vb]
                    slo = ((lo[0] + lo[1]) + (lo[2] + lo[3])) + \
                          ((lo[4] + lo[5]) + (lo[6] + lo[7]))
                    shi = ((hi[0] + hi[1]) + (hi[2] + hi[3])) + \
                          ((hi[4] + hi[5]) + (hi[6] + hi[7]))
                    acc[par][g, pl.ds(col, 16)] = slo * (1.0 / _K)
                    acc[par][g, pl.ds(col + _FP, 16)] = shi * (1.0 / _K)

            @pl.when(c + 2 < _NCHUNK)
            def _():
                pltpu.async_copy(
                    table_hbm.at[idx_v.at[c + 2]], rows[par], gsem[par])

            pltpu.async_copy(
                acc[par], out_hbm.at[pl.ds(base + c * _G, _G)], wsem[par])
        return carry

    lax.fori_loop(0, _NCHUNK // 2, outer, 0)

    pltpu.make_async_copy(acc0, out_hbm.at[pl.ds(0, _G)], ws0).wait()
    pltpu.make_async_copy(acc1, out_hbm.at[pl.ds(0, _G)], ws1).wait()


@functools.lru_cache(maxsize=1)
def _make_gather_mean():
    return functools.partial(
        pl.kernel,
        mesh=plsc.VectorSubcoreMesh(core_axis_name="c", subcore_axis_name="s"),
        compiler_params=pltpu.CompilerParams(needs_layout_passes=False),
        out_type=jax.ShapeDtypeStruct((_B * _N, _F), jnp.float32),
        scratch_types=[
            pltpu.VMEM((_NCHUNK, _G * _K), jnp.int32),
            pltpu.VMEM((_G * _K, _FP), jnp.int32),
            pltpu.VMEM((_G * _K, _FP), jnp.int32),
            pltpu.VMEM((_G, _F), jnp.float32),
            pltpu.VMEM((_G, _F), jnp.float32),
            pltpu.SemaphoreType.DMA,
            pltpu.SemaphoreType.DMA,
            pltpu.SemaphoreType.DMA,
            pltpu.SemaphoreType.DMA,
        ],
    )(_gather_mean_body)


def _fused_mm_bn_body(feat_ref, m_ref, w_ref, bias_ref, g_ref, be_ref,
                      out_ref, trans_s, feat_s, s1_s, s2_s):
    p = pl.program_id(0)
    bi = pl.program_id(1)
    j = pl.program_id(2)
    sl = pl.ds(j * _RBLK, _RBLK)

    @pl.when(p == 0)
    def _():
        f = feat_ref[0]
        feat_s[bi, sl, :] = f
        lap = f - m_ref[0]
        t = lax.dot_general(
            lap, w_ref[...], (((1,), (1,)), ((), ())),
            preferred_element_type=jnp.float32) + bias_ref[...]
        trans_s[bi, sl, :] = t
        rs1 = jnp.sum(t, axis=1, keepdims=True)
        rs2 = jnp.sum(t * t, axis=1, keepdims=True)

        @pl.when(bi == 0)
        def _():
            s1_s[sl] = rs1
            s2_s[sl] = rs2

        @pl.when(bi != 0)
        def _():
            s1_s[sl] = s1_s[sl] + rs1
            s2_s[sl] = s2_s[sl] + rs2

    @pl.when(p == 1)
    def _():
        cnt = float(_B * _F)
        mean = s1_s[sl] / cnt
        var = s2_s[sl] / cnt - mean * mean
        inv = lax.rsqrt(var + _EPS)
        t = trans_s[bi, sl, :]
        y = (t - mean) * inv * g_ref[...] + be_ref[...]
        out_ref[0] = feat_s[bi, sl, :] + jnp.maximum(y, 0.0)


def _fused_mm_bn_call(feat, m, w, bias, gamma, beta):
    def m_map(p, i, j):
        return (jnp.where(p == 0, i, 0), jnp.where(p == 0, j, 0), 0)

    def out_map(p, i, j):
        return (jnp.where(p == 0, 0, i), jnp.where(p == 0, 0, j), 0)

    return pl.pallas_call(
        _fused_mm_bn_body,
        grid=(2, _B, _N // _RBLK),
        in_specs=[
            pl.BlockSpec((1, _RBLK, _F), m_map),
            pl.BlockSpec((1, _RBLK, _F), m_map),
            pl.BlockSpec((_F, _F), lambda p, i, j: (0, 0)),
            pl.BlockSpec((1, _F), lambda p, i, j: (0, 0)),
            pl.BlockSpec((_RBLK, 1), lambda p, i, j: (j, 0)),
            pl.BlockSpec((_RBLK, 1), lambda p, i, j: (j, 0)),
        ],
        out_specs=pl.BlockSpec((1, _RBLK, _F), out_map),
        out_shape=jax.ShapeDtypeStruct((_B, _N, _F), jnp.float32),
        scratch_shapes=[
            pltpu.VMEM((_B, _N, _F), jnp.float32),
            pltpu.VMEM((_B, _N, _F), jnp.float32),
            pltpu.VMEM((_N, 1), jnp.float32),
            pltpu.VMEM((_N, 1), jnp.float32),
        ],
    )(feat, m, w, bias, gamma, beta)


def kernel(xyz, feat, W, b, gamma, beta):
    xyn = xyz[:, :, :2]
    xyt = jnp.transpose(xyn, (0, 2, 1))
    idx, featT = _topk_call(xyn, xyt, feat)

    f16 = featT.reshape(_B * _N, _F)
    tbl = lax.bitcast_convert_type(
        jnp.stack([f16[:, :_FP], f16[:, _FP:]], axis=-1), jnp.int32)

    cols = idx.reshape(_B, _K, _N // _K, _K).transpose(0, 2, 3, 1)
    cols = cols.reshape(_B, _N, _K)
    cols = cols + (jnp.arange(_B, dtype=jnp.int32) * _N)[:, None, None]
    idx_sc = cols.reshape(_NW, _NCHUNK, _G * _K)

    m = _make_gather_mean()(tbl, idx_sc)
    m = m.reshape(_B, _N, _F)

    return _fused_mm_bn_call(feat, m, W, b.reshape(1, _F),
                             gamma.reshape(_N, 1), beta.reshape(_N, 1))
